# Initial kernel scaffold; baseline (speedup 1.0000x reference)
import jax
import jax.numpy as jnp
from jax.experimental import pallas as pl


def kernel(x, edge_index, W1, b1, W2, b2, W3, b3, Wr, br, g1, be1, g2, be2):
    raise NotImplementedError("write your pallas kernel here")


# speedup vs baseline: 2.7489x; 2.7489x over previous
import functools

import jax
import jax.numpy as jnp
from jax.experimental import pallas as pl
from jax.experimental.pallas import tpu as pltpu

N = 50000
E = 800000
EPS = 1e-5
B = 2000


def _dense1_body(x_ref, dis_ref, y1_ref):
    dis = dis_ref[...]
    x = x_ref[...]
    y1 = dis * x
    y1_ref[...] = jnp.pad(y1, ((0, 0), (0, 12)))


def _dense1(x, dis):
    return pl.pallas_call(
        _dense1_body,
        grid=(N // B,),
        in_specs=[
            pl.BlockSpec((B, 20), lambda i: (i, 0)),
            pl.BlockSpec((B, 1), lambda i: (i, 0)),
        ],
        out_specs=pl.BlockSpec((B, 32), lambda i: (i, 0)),
        out_shape=jax.ShapeDtypeStruct((N, 32), jnp.float32),
    )(x, dis)


def _layer_norm(z, g, b):
    mu = jnp.mean(z, axis=-1, keepdims=True)
    var = jnp.mean((z - mu) ** 2, axis=-1, keepdims=True)
    return (z - mu) * jax.lax.rsqrt(var + EPS) * g + b


def _dense2_body(x_ref, y1_ref, s1_ref, dis_ref, w1t_ref, wrt_ref, bias_ref,
                 h_ref, y2_ref):
    dis = dis_ref[...]
    agg1 = dis * (s1_ref[...] + y1_ref[...])
    h1 = jnp.dot(agg1, w1t_ref[...], preferred_element_type=jnp.float32)
    res = jnp.dot(x_ref[...], wrt_ref[...], preferred_element_type=jnp.float32)
    b1 = bias_ref[0:1, :]
    br = bias_ref[1:2, :]
    g1 = bias_ref[2:3, :]
    be1 = bias_ref[3:4, :]
    z = h1 + b1 + res + br
    h = jax.nn.relu(_layer_norm(z, g1, be1))
    h_ref[...] = h
    y2_ref[...] = dis * h


def _dense2(x, y1p, s1, dis, w1t, wrt, bias):
    return pl.pallas_call(
        _dense2_body,
        grid=(N // B,),
        in_specs=[
            pl.BlockSpec((B, 20), lambda i: (i, 0)),
            pl.BlockSpec((B, 32), lambda i: (i, 0)),
            pl.BlockSpec((B, 32), lambda i: (i, 0)),
            pl.BlockSpec((B, 1), lambda i: (i, 0)),
            pl.BlockSpec((32, 128), lambda i: (0, 0)),
            pl.BlockSpec((20, 128), lambda i: (0, 0)),
            pl.BlockSpec((4, 128), lambda i: (0, 0)),
        ],
        out_specs=[
            pl.BlockSpec((B, 128), lambda i: (i, 0)),
            pl.BlockSpec((B, 128), lambda i: (i, 0)),
        ],
        out_shape=[
            jax.ShapeDtypeStruct((N, 128), jnp.float32),
            jax.ShapeDtypeStruct((N, 128), jnp.float32),
        ],
    )(x, y1p, s1, dis, w1t, wrt, bias)


def _dense3_body(h_ref, y2_ref, s2_ref, dis_ref, cw_ref, w2t_ref, bias_ref,
                 w3t_ref, b3_ref, out_ref, acc_ref):
    i = pl.program_id(0)
    dis = dis_ref[...]
    agg2 = dis * (s2_ref[...] + y2_ref[...])
    t = jnp.dot(agg2, w2t_ref[...], preferred_element_type=jnp.float32)
    b2 = bias_ref[0:1, :]
    g2 = bias_ref[1:2, :]
    be2 = bias_ref[2:3, :]
    h2 = jax.nn.relu(_layer_norm(t + b2 + h_ref[...], g2, be2))
    w = dis * (cw_ref[...] + dis)
    part = jnp.sum(w * h2, axis=0, keepdims=True)

    @pl.when(i == 0)
    def _():
        acc_ref[...] = jnp.zeros_like(acc_ref)

    acc_ref[...] += part

    @pl.when(i == pl.num_programs(0) - 1)
    def _():
        pooled = acc_ref[...] * (1.0 / N)
        out_ref[...] = jnp.dot(pooled, w3t_ref[...],
                               preferred_element_type=jnp.float32) + b3_ref[...]


def _dense3(h, y2, s2, dis, c, w2t, bias, w3t, b3):
    return pl.pallas_call(
        _dense3_body,
        grid=(N // B,),
        in_specs=[
            pl.BlockSpec((B, 128), lambda i: (i, 0)),
            pl.BlockSpec((B, 128), lambda i: (i, 0)),
            pl.BlockSpec((B, 128), lambda i: (i, 0)),
            pl.BlockSpec((B, 1), lambda i: (i, 0)),
            pl.BlockSpec((B, 1), lambda i: (i, 0)),
            pl.BlockSpec((128, 128), lambda i: (0, 0)),
            pl.BlockSpec((3, 128), lambda i: (0, 0)),
            pl.BlockSpec((128, 256), lambda i: (0, 0)),
            pl.BlockSpec((1, 256), lambda i: (0, 0)),
        ],
        out_specs=pl.BlockSpec((1, 256), lambda i: (0, 0)),
        out_shape=jax.ShapeDtypeStruct((1, 256), jnp.float32),
        scratch_shapes=[pltpu.VMEM((1, 128), jnp.float32)],
    )(h, y2, s2, dis, c, w2t, bias, w3t, b3)


def kernel(x, edge_index, W1, b1, W2, b2, W3, b3, Wr, br, g1, be1, g2, be2):
    src = edge_index[0]
    dst = edge_index[1]

    deg = jnp.ones((N,), jnp.float32).at[dst].add(1.0)
    dis = jax.lax.rsqrt(deg)
    dis2d = dis[:, None]

    c = jnp.zeros((N,), jnp.float32).at[src].add(dis[dst])

    y1p = _dense1(x, dis2d)
    s1 = jnp.zeros((N, 32), jnp.float32).at[dst].add(y1p[src])

    w1t = jnp.pad(W1.T, ((0, 12), (0, 0)))
    bias2 = jnp.stack([b1, br, g1, be1])
    h, y2 = _dense2(x, y1p, s1, dis2d, w1t, Wr.T, bias2)

    s2 = jnp.zeros((N, 128), jnp.float32).at[dst].add(y2[src])

    bias3 = jnp.stack([b2, g2, be2])
    out = _dense3(h, y2, s2, dis2d, c[:, None], W2.T, bias3, W3.T, b3[None, :])
    return out
